# baseline (speedup 1.0000x reference)
import functools

import jax
import jax.numpy as jnp
from jax.experimental import pallas as pl
from jax.experimental.pallas import tpu as pltpu


@functools.lru_cache(maxsize=None)
def _make_row_gather(V: int):
    def body(tok_ref, cooc_ref, out_ref, sem):
        tok = tok_ref[0]
        pltpu.make_async_copy(
            cooc_ref.at[pl.ds(tok, 1)], out_ref.at[0], sem
        ).start()
        pltpu.make_async_copy(
            cooc_ref.at[pl.ds(tok, 1)], out_ref.at[0], sem
        ).wait()

    grid_spec = pltpu.PrefetchScalarGridSpec(
        num_scalar_prefetch=1,
        grid=(1,),
        in_specs=[pl.BlockSpec(memory_space=pltpu.MemorySpace.HBM)],
        out_specs=pl.BlockSpec(memory_space=pltpu.MemorySpace.HBM),
        scratch_shapes=[pltpu.SemaphoreType.DMA],
    )
    return pl.pallas_call(
        body,
        grid_spec=grid_spec,
        out_shape=jax.ShapeDtypeStruct((1, 1, V), jnp.float32),
        compiler_params=pltpu.CompilerParams(allow_input_fusion=[True, False]),
    )


def kernel(_, decoder_input_ids, cooc):
    V = cooc.shape[0]
    tok = decoder_input_ids[0, -1:].astype(jnp.int32)
    return _make_row_gather(V)(tok, cooc)


# speedup vs baseline: 1.2082x; 1.2082x over previous
import functools

import jax
import jax.numpy as jnp
from jax.experimental import pallas as pl
from jax.experimental.pallas import tpu as pltpu


@functools.lru_cache(maxsize=None)
def _make_row_gather(V: int, L: int):
    NB = L // 128

    def body(ids_ref, cooc_ref, out_ref, sem):
        tok = ids_ref[0, 127]
        pltpu.make_async_copy(
            cooc_ref.at[pl.ds(tok, 1)], out_ref.at[0], sem
        ).start()
        pltpu.make_async_copy(
            cooc_ref.at[pl.ds(tok, 1)], out_ref.at[0], sem
        ).wait()

    return pl.pallas_call(
        body,
        grid=(1,),
        in_specs=[
            pl.BlockSpec((1, 128), lambda i: (0, NB - 1),
                         memory_space=pltpu.MemorySpace.SMEM),
            pl.BlockSpec(memory_space=pltpu.MemorySpace.HBM),
        ],
        out_specs=pl.BlockSpec(memory_space=pltpu.MemorySpace.HBM),
        scratch_shapes=[pltpu.SemaphoreType.DMA],
        out_shape=jax.ShapeDtypeStruct((1, 1, V), jnp.float32),
    )


def kernel(_, decoder_input_ids, cooc):
    V = cooc.shape[0]
    L = decoder_input_ids.shape[1]
    ids = decoder_input_ids.astype(jnp.int32)
    return _make_row_gather(V, L)(ids, cooc)


# speedup vs baseline: 1.2328x; 1.0203x over previous
import functools

import jax
import jax.numpy as jnp
from jax.experimental import pallas as pl
from jax.experimental.pallas import tpu as pltpu


@functools.lru_cache(maxsize=None)
def _make_row_gather(V: int, L: int):
    NB = L // 128

    H = 16000

    def body(ids_ref, cooc_ref, out_ref, sem, sem2):
        tok = ids_ref[0, 127]
        a = pltpu.make_async_copy(
            cooc_ref.at[pl.ds(tok, 1), pl.ds(0, H)],
            out_ref.at[0, :, pl.ds(0, H)], sem)
        b = pltpu.make_async_copy(
            cooc_ref.at[pl.ds(tok, 1), pl.ds(H, V - H)],
            out_ref.at[0, :, pl.ds(H, V - H)], sem2)
        a.start()
        b.start()
        a.wait()
        b.wait()

    return pl.pallas_call(
        body,
        grid=(1,),
        in_specs=[
            pl.BlockSpec((1, 128), lambda i: (0, NB - 1),
                         memory_space=pltpu.MemorySpace.SMEM),
            pl.BlockSpec(memory_space=pltpu.MemorySpace.HBM),
        ],
        out_specs=pl.BlockSpec(memory_space=pltpu.MemorySpace.HBM),
        scratch_shapes=[pltpu.SemaphoreType.DMA, pltpu.SemaphoreType.DMA],
        out_shape=jax.ShapeDtypeStruct((1, 1, V), jnp.float32),
    )


def kernel(_, decoder_input_ids, cooc):
    V = cooc.shape[0]
    L = decoder_input_ids.shape[1]
    ids = decoder_input_ids.astype(jnp.int32)
    return _make_row_gather(V, L)(ids, cooc)
